# Initial kernel scaffold; baseline (speedup 1.0000x reference)
import functools

import jax
import jax.numpy as jnp
from jax import lax
from jax.experimental import pallas as pl
from jax.experimental.pallas import tpu as pltpu
from jax.experimental.pallas import tpu_sc as plsc

MAX_POS = 1024
DIM = 64

_INFO = plsc.get_sparse_core_info()
NC, NS, L = _INFO.num_cores, _INFO.num_subcores, _INFO.num_lanes
NW = NC * NS

CHUNK = 128
GBLK = 128


def _table_body(x_ref, y_ref, w_ref, b_ref, t_ref):
    wt = w_ref[...].T
    bias = b_ref[...] * 0.25
    t_ref[0:MAX_POS, :] = (
        jnp.dot(x_ref[...], wt, preferred_element_type=jnp.float32) + bias
    )
    t_ref[MAX_POS : 2 * MAX_POS, :] = (
        jnp.dot(y_ref[...], wt, preferred_element_type=jnp.float32) + bias
    )


def _build_table(x_table, y_table, W, b):
    return pl.pallas_call(
        _table_body,
        out_shape=jax.ShapeDtypeStruct((2 * MAX_POS, DIM), jnp.float32),
    )(x_table, y_table, W, b.reshape(1, DIM))


def _sc_body(rows_total, t_hbm, bb_hbm, out_hbm, bb_v, idx_v, rows_v, out_v, sem):
    rw = rows_total // NW
    n_chunks = rw // CHUNK
    wid = lax.axis_index("s") * NC + lax.axis_index("c")
    base_row = wid * rw

    offs = (lax.iota(jnp.int32, L) % 2) * MAX_POS

    def chunk_body(c, carry):
        row0 = base_row + c * CHUNK
        pltpu.sync_copy(bb_hbm.at[pl.ds(row0 * 4, 4 * CHUNK)], bb_v)
        for j in range(4 * CHUNK // GBLK):
            for i in range(GBLK // L):
                v = bb_v[pl.ds(j * GBLK + i * L, L)]
                f = jnp.clip(v * float(MAX_POS), 0.0, float(MAX_POS - 1))
                idx_v[j, pl.ds(i * L, L)] = f.astype(jnp.int32) + offs
        copies = [
            pltpu.async_copy(
                t_hbm.at[idx_v.at[j]], rows_v.at[pl.ds(j * GBLK, GBLK)], sem
            )
            for j in range(4 * CHUNK // GBLK)
        ]
        for cp in copies:
            cp.wait()

        def sum_body(r, carry2):
            for d in range(DIM // L):
                ds = pl.ds(d * L, L)
                s = (
                    rows_v[4 * r, ds]
                    + rows_v[4 * r + 1, ds]
                    + rows_v[4 * r + 2, ds]
                    + rows_v[4 * r + 3, ds]
                )
                out_v[r, ds] = jnp.maximum(s, 0.0)
            return carry2

        lax.fori_loop(0, CHUNK, sum_body, 0)
        pltpu.sync_copy(out_v, out_hbm.at[pl.ds(row0, CHUNK)])
        return carry

    lax.fori_loop(0, n_chunks, chunk_body, 0)


def _lookup(t, bb_flat, rows_total):
    mesh = plsc.VectorSubcoreMesh(core_axis_name="c", subcore_axis_name="s")
    f = pl.kernel(
        functools.partial(_sc_body, rows_total),
        out_type=jax.ShapeDtypeStruct((rows_total, DIM), jnp.float32),
        mesh=mesh,
        scratch_types=[
            pltpu.VMEM((4 * CHUNK,), jnp.float32),
            pltpu.VMEM((4 * CHUNK // GBLK, GBLK), jnp.int32),
            pltpu.VMEM((4 * CHUNK, DIM), jnp.float32),
            pltpu.VMEM((CHUNK, DIM), jnp.float32),
            pltpu.SemaphoreType.DMA,
        ],
    )
    return f(t, bb_flat)


def kernel(gt_bboxes, x_table, y_table, W, b):
    B, N, _ = gt_bboxes.shape
    rows_total = B * N
    t = _build_table(x_table, y_table, W, b)
    bb_flat = gt_bboxes.reshape(rows_total * 4)
    out = _lookup(t, bb_flat, rows_total)
    return out.reshape(B, N, DIM)


# speedup vs baseline: 6.1964x; 6.1964x over previous
import functools

import jax
import jax.numpy as jnp
from jax import lax
from jax.experimental import pallas as pl
from jax.experimental.pallas import tpu as pltpu
from jax.experimental.pallas import tpu_sc as plsc

MAX_POS = 1024
DIM = 64

try:
    _INFO = plsc.get_sparse_core_info()
    NC, NS, L = _INFO.num_cores, _INFO.num_subcores, _INFO.num_lanes
except Exception:
    NC, NS, L = 2, 16, 16
NW = NC * NS

CHUNK = 128
GBLK = 128


def _table_body(x_ref, y_ref, w_ref, b_ref, t_ref):
    wt = w_ref[...].T
    bias = b_ref[...] * 0.25
    t_ref[0:MAX_POS, :] = (
        jnp.dot(x_ref[...], wt, preferred_element_type=jnp.float32) + bias
    )
    t_ref[MAX_POS : 2 * MAX_POS, :] = (
        jnp.dot(y_ref[...], wt, preferred_element_type=jnp.float32) + bias
    )


def _build_table(x_table, y_table, W, b):
    return pl.pallas_call(
        _table_body,
        out_shape=jax.ShapeDtypeStruct((2 * MAX_POS, DIM), jnp.float32),
    )(x_table, y_table, W, b.reshape(1, DIM))


def _sc_body(rows_total, t_hbm, bb_hbm, out_hbm, bb_v, idx_v, rows_v, out_v, sem):
    rw = rows_total // NW
    n_chunks = rw // CHUNK
    wid = lax.axis_index("s") * NC + lax.axis_index("c")
    base_row = wid * rw

    offs = (lax.iota(jnp.int32, L) % 2) * MAX_POS

    def chunk_body(c, carry):
        row0 = base_row + c * CHUNK
        pltpu.sync_copy(bb_hbm.at[pl.ds(row0 * 4, 4 * CHUNK)], bb_v)
        for j in range(4 * CHUNK // GBLK):
            for i in range(GBLK // L):
                v = bb_v[pl.ds(j * GBLK + i * L, L)]
                f = jnp.clip(v * float(MAX_POS), 0.0, float(MAX_POS - 1))
                idx_v[j, pl.ds(i * L, L)] = f.astype(jnp.int32) + offs
        copies = [
            pltpu.async_copy(
                t_hbm.at[idx_v.at[j]], rows_v.at[pl.ds(j * GBLK, GBLK)], sem
            )
            for j in range(4 * CHUNK // GBLK)
        ]
        for cp in copies:
            cp.wait()

        def sum_body(r, carry2):
            for d in range(DIM // L):
                ds = pl.ds(d * L, L)
                s = (
                    rows_v[4 * r, ds]
                    + rows_v[4 * r + 1, ds]
                    + rows_v[4 * r + 2, ds]
                    + rows_v[4 * r + 3, ds]
                )
                out_v[r, ds] = jnp.maximum(s, 0.0)
            return carry2

        lax.fori_loop(0, CHUNK, sum_body, 0)
        pltpu.sync_copy(out_v, out_hbm.at[pl.ds(row0, CHUNK)])
        return carry

    lax.fori_loop(0, n_chunks, chunk_body, 0)


def _lookup(t, bb_flat, rows_total):
    mesh = plsc.VectorSubcoreMesh(
        core_axis_name="c", subcore_axis_name="s", num_cores=NC, num_subcores=NS
    )
    f = pl.kernel(
        functools.partial(_sc_body, rows_total),
        out_type=jax.ShapeDtypeStruct((rows_total, DIM), jnp.float32),
        mesh=mesh,
        scratch_types=[
            pltpu.VMEM((4 * CHUNK,), jnp.float32),
            pltpu.VMEM((4 * CHUNK // GBLK, GBLK), jnp.int32),
            pltpu.VMEM((4 * CHUNK, DIM), jnp.float32),
            pltpu.VMEM((CHUNK, DIM), jnp.float32),
            pltpu.SemaphoreType.DMA,
        ],
        compiler_params=pltpu.CompilerParams(use_tc_tiling_on_sc=False),
    )
    return f(t, bb_flat)


def kernel(gt_bboxes, x_table, y_table, W, b):
    B, N, _ = gt_bboxes.shape
    rows_total = B * N
    t = _build_table(x_table, y_table, W, b)
    bb_flat = gt_bboxes.reshape(rows_total * 4)
    out = _lookup(t, bb_flat, rows_total)
    return out.reshape(B, N, DIM)
